# baseline (speedup 1.0000x reference)
import functools

import jax
import jax.numpy as jnp
from jax.experimental import pallas as pl
from jax.experimental.pallas import tpu as pltpu

_SMOOTH = 1e-5
_INV_SQRT2 = 0.7071067811865476


def _stencil_kernel(thr_ref, aw_ref, x_ref, o_ref,
                    g_scr, tail_scr, s_ref, inter_ref, mask_ref, dinv_ref,
                    w_ref, copy_sem, *, n, c, k):
    u = pl.program_id(0)
    nu = n // k
    kc, h, w = x_ref.shape
    hb = 16 if h % 16 == 0 else h
    g = min(4, k)
    thr = thr_ref[0]
    aw = aw_ref[0]
    base = k * u


    window_copy = pltpu.make_async_copy(x_ref, g_scr.at[u % 2], copy_sem)

    @pl.when(u <= nu - 1)
    def _():
        window_copy.start()

    prev_ref = g_scr.at[(u + 1) % 2]

    def tile(ref, j, h0):
        return ref[j * c:(j + 1) * c, h0:h0 + hb, :]

    s_tot = [None] * k
    i_tot = [None] * k
    sub = 8 if (c * hb) % 8 == 0 else 1

    def psum(t3):
        return jnp.sum(jnp.reshape(t3, (-1, sub, w)), axis=0)

    def put_s(i, v):
        @pl.when((i >= 0) & (i <= n - 1))
        def _():
            s_ref[i] = v

    def put_inter(i, v):
        @pl.when((i >= 0) & (i <= n - 2))
        def _():
            inter_ref[i] = v

    def put_mask(i):
        @pl.when((i >= 0) & (i <= n - 2))
        def _():
            dice = ((2.0 * inter_ref[i] + _SMOOTH)
                    / (s_ref[i] + s_ref[i + 1] + _SMOOTH))
            mask_ref[i] = jnp.where((dice > thr) & (dice < 1.0), 1.0, 0.0)

    def getm(i):
        ok = (i >= 0) & (i <= n - 2)
        return jnp.where(ok, mask_ref[jnp.clip(i, 0, n - 2)], 0.0)

    def put_dinv(m):
        @pl.when((m >= 0) & (m <= n - 1))
        def _():
            deg = getm(m - 1) + getm(m)
            dinv_ref[m] = jnp.where(deg > 1.5, _INV_SQRT2,
                                    jnp.where(deg > 0.5, 1.0, 0.0))

    def put_w(i):
        @pl.when((i >= 0) & (i <= n - 2))
        def _():
            w_ref[i] = aw * getm(i) * dinv_ref[i] * dinv_ref[i + 1]

    for g0 in range(0, k, g):
        s_acc = [jnp.zeros((sub, w), jnp.float32) for _ in range(g)]
        i_acc = [jnp.zeros((sub, w), jnp.float32) for _ in range(g)]
        for h0 in range(0, h, hb):
            if g0 == 0:
                pbl = (tile(prev_ref, k - 1, h0) > 0.0).astype(jnp.float32)
            else:
                pbl = (tile(x_ref, g0 - 1, h0) > 0.0).astype(jnp.float32)
            for t in range(g):
                v = tile(x_ref, g0 + t, h0)
                pb = (v > 0.0).astype(jnp.float32)
                s_acc[t] = s_acc[t] + psum(pb)
                i_acc[t] = i_acc[t] + psum(pb * pbl)
                pbl = pb
        for t in range(g):
            s_tot[g0 + t] = jnp.sum(s_acc[t])
            i_tot[g0 + t] = jnp.sum(i_acc[t])
            put_s(base + g0 + t, s_tot[g0 + t])
            put_inter(base + g0 + t - 1, i_tot[g0 + t])
        for off in range(g0 - 1, g0 + g - 1):
            put_mask(base + off)
        for off in range(g0 - 1, g0 + g - 1):
            put_dinv(base + off)
        for off in range(g0 - 2, g0 + g - 2):
            put_w(base + off)

    def getw(i):
        ok = (i >= 0) & (i <= n - 2)
        return jnp.where(ok, w_ref[jnp.clip(i, 0, n - 2)], 0.0)

    m0 = base - k
    wl = [getw(m0 + j - 1) for j in range(k)]
    wr = [getw(m0 + j) for j in range(k)]

    @pl.when(u >= 1)
    def _():
        for h0 in range(0, h, hb):
            r_prev = jnp.maximum(tail_scr[:, h0:h0 + hb, :], 0.0)
            v_cur = tile(prev_ref, 0, h0)
            r_cur = jnp.maximum(v_cur, 0.0)
            for j in range(k):
                if j <= k - 2:
                    v_next = tile(prev_ref, j + 1, h0)
                else:
                    v_next = tile(x_ref, 0, h0)
                r_next = jnp.maximum(v_next, 0.0)
                o_ref[j * c:(j + 1) * c, h0:h0 + hb, :] = v_cur + (
                    r_prev * wl[j] + r_next * wr[j])
                r_prev = r_cur
                v_cur = v_next
                r_cur = r_next

    @pl.when(u >= 1)
    def _():
        tail_scr[...] = prev_ref[(k - 1) * c:, :, :]

    @pl.when(u == 0)
    def _():
        tail_scr[...] = jnp.zeros_like(tail_scr)

    @pl.when(u <= nu - 1)
    def _():
        window_copy.wait()


def kernel(x, adaptive_weight, similarity_threshold):
    b, c, d, h, w = x.shape
    n = b * d
    k = 16
    while n % k != 0 or n // k < 2:
        k //= 2
    nu = n // k
    gf = jnp.reshape(x, (b * c * d, h, w))
    thr = jnp.reshape(similarity_threshold, (1,)).astype(jnp.float32)
    aw = jnp.reshape(adaptive_weight, (1,)).astype(jnp.float32)

    out = pl.pallas_call(
        functools.partial(_stencil_kernel, n=n, c=c, k=k),
        grid=(nu + 1,),
        in_specs=[
            pl.BlockSpec(memory_space=pltpu.SMEM),
            pl.BlockSpec(memory_space=pltpu.SMEM),
            pl.BlockSpec((k * c, h, w),
                         lambda u: (jnp.minimum(u, nu - 1), 0, 0)),
        ],
        out_specs=pl.BlockSpec((k * c, h, w),
                               lambda u: (jnp.clip(u - 1, 0, nu - 1), 0, 0)),
        out_shape=jax.ShapeDtypeStruct((b * c * d, h, w), jnp.float32),
        scratch_shapes=[
            pltpu.VMEM((2, k * c, h, w), jnp.float32),
            pltpu.VMEM((c, h, w), jnp.float32),
            pltpu.SMEM((n + 8,), jnp.float32),
            pltpu.SMEM((n + 8,), jnp.float32),
            pltpu.SMEM((n + 8,), jnp.float32),
            pltpu.SMEM((n + 8,), jnp.float32),
            pltpu.SMEM((n + 8,), jnp.float32),
            pltpu.SemaphoreType.DMA,
        ],
    )(thr, aw, gf)

    return jnp.reshape(out, (b, c, d, h, w))


# speedup vs baseline: 1.1026x; 1.1026x over previous
import functools

import jax
import jax.numpy as jnp
from jax.experimental import pallas as pl
from jax.experimental.pallas import tpu as pltpu

_SMOOTH = 1e-5
_INV_SQRT2 = 0.7071067811865476


def _stencil_kernel(thr_ref, aw_ref, x_ref, o_ref,
                    g_scr, tail_scr, s_ref, inter_ref, mask_ref, dinv_ref,
                    w_ref, copy_sem, *, n, c, k):
    u = pl.program_id(0)
    nu = n // k
    kc, h, w = x_ref.shape
    hb = 16 if h % 16 == 0 else h
    g = min(4, k)
    thr = thr_ref[0]
    aw = aw_ref[0]
    base = k * u


    window_copy = pltpu.make_async_copy(x_ref, g_scr.at[u % 2], copy_sem)

    @pl.when(u <= nu - 1)
    def _():
        window_copy.start()

    prev_ref = g_scr.at[(u + 1) % 2]

    def tile(ref, j, h0):
        return ref[j * c:(j + 1) * c, h0:h0 + hb, :]

    s_tot = [None] * k
    i_tot = [None] * k
    sub = 8 if (c * hb) % 8 == 0 else 1

    def psum(t3):
        return jnp.sum(jnp.reshape(t3, (-1, sub, w)), axis=0)

    cl = n + k

    def put_s(i, v):
        s_ref[jnp.clip(i, 0, cl)] = v

    def put_inter(i, v):
        inter_ref[jnp.clip(i, 0, cl)] = v

    def put_mask(i):
        ic = jnp.clip(i, 0, cl)
        dice = ((2.0 * inter_ref[ic] + _SMOOTH)
                / (s_ref[ic] + s_ref[jnp.clip(i + 1, 0, cl)] + _SMOOTH))
        mask_ref[ic] = jnp.where((dice > thr) & (dice < 1.0), 1.0, 0.0)

    def getm(i):
        ok = (i >= 0) & (i <= n - 2)
        return jnp.where(ok, mask_ref[jnp.clip(i, 0, n - 2)], 0.0)

    def put_dinv(m):
        deg = getm(m - 1) + getm(m)
        dinv_ref[jnp.clip(m, 0, cl)] = jnp.where(
            deg > 1.5, _INV_SQRT2, jnp.where(deg > 0.5, 1.0, 0.0))

    def put_w(i):
        ic = jnp.clip(i, 0, cl)
        w_ref[ic] = (aw * getm(i) * dinv_ref[ic]
                     * dinv_ref[jnp.clip(i + 1, 0, cl)])

    for g0 in range(0, k, g):
        s_acc = [jnp.zeros((sub, w), jnp.float32) for _ in range(g)]
        i_acc = [jnp.zeros((sub, w), jnp.float32) for _ in range(g)]
        for h0 in range(0, h, hb):
            if g0 == 0:
                pbl = (tile(prev_ref, k - 1, h0) > 0.0).astype(jnp.float32)
            else:
                pbl = (tile(x_ref, g0 - 1, h0) > 0.0).astype(jnp.float32)
            for t in range(g):
                v = tile(x_ref, g0 + t, h0)
                pb = (v > 0.0).astype(jnp.float32)
                s_acc[t] = s_acc[t] + psum(pb)
                i_acc[t] = i_acc[t] + psum(pb * pbl)
                pbl = pb
        for t in range(g):
            s_tot[g0 + t] = jnp.sum(s_acc[t])
            i_tot[g0 + t] = jnp.sum(i_acc[t])
            put_s(base + g0 + t, s_tot[g0 + t])
            put_inter(base + g0 + t - 1, i_tot[g0 + t])
        for off in range(g0 - 1, g0 + g - 1):
            put_mask(base + off)
        for off in range(g0 - 1, g0 + g - 1):
            put_dinv(base + off)
        for off in range(g0 - 2, g0 + g - 2):
            put_w(base + off)

    def getw(i):
        ok = (i >= 0) & (i <= n - 2)
        return jnp.where(ok, w_ref[jnp.clip(i, 0, n - 2)], 0.0)

    m0 = base - k
    wl = [getw(m0 + j - 1) for j in range(k)]
    wr = [getw(m0 + j) for j in range(k)]

    @pl.when(u >= 1)
    def _():
        for h0 in range(0, h, hb):
            r_prev = jnp.maximum(tail_scr[:, h0:h0 + hb, :], 0.0)
            v_cur = tile(prev_ref, 0, h0)
            r_cur = jnp.maximum(v_cur, 0.0)
            for j in range(k):
                if j <= k - 2:
                    v_next = tile(prev_ref, j + 1, h0)
                else:
                    v_next = tile(x_ref, 0, h0)
                r_next = jnp.maximum(v_next, 0.0)
                o_ref[j * c:(j + 1) * c, h0:h0 + hb, :] = v_cur + (
                    r_prev * wl[j] + r_next * wr[j])
                r_prev = r_cur
                v_cur = v_next
                r_cur = r_next

    @pl.when(u >= 1)
    def _():
        tail_scr[...] = prev_ref[(k - 1) * c:, :, :]

    @pl.when(u == 0)
    def _():
        tail_scr[...] = jnp.zeros_like(tail_scr)

    @pl.when(u <= nu - 1)
    def _():
        window_copy.wait()


def kernel(x, adaptive_weight, similarity_threshold):
    b, c, d, h, w = x.shape
    n = b * d
    k = 16
    while n % k != 0 or n // k < 2:
        k //= 2
    nu = n // k
    gf = jnp.reshape(x, (b * c * d, h, w))
    thr = jnp.reshape(similarity_threshold, (1,)).astype(jnp.float32)
    aw = jnp.reshape(adaptive_weight, (1,)).astype(jnp.float32)

    out = pl.pallas_call(
        functools.partial(_stencil_kernel, n=n, c=c, k=k),
        grid=(nu + 1,),
        in_specs=[
            pl.BlockSpec(memory_space=pltpu.SMEM),
            pl.BlockSpec(memory_space=pltpu.SMEM),
            pl.BlockSpec((k * c, h, w),
                         lambda u: (jnp.minimum(u, nu - 1), 0, 0)),
        ],
        out_specs=pl.BlockSpec((k * c, h, w),
                               lambda u: (jnp.clip(u - 1, 0, nu - 1), 0, 0)),
        out_shape=jax.ShapeDtypeStruct((b * c * d, h, w), jnp.float32),
        scratch_shapes=[
            pltpu.VMEM((2, k * c, h, w), jnp.float32),
            pltpu.VMEM((c, h, w), jnp.float32),
            pltpu.SMEM((n + k + 8,), jnp.float32),
            pltpu.SMEM((n + k + 8,), jnp.float32),
            pltpu.SMEM((n + k + 8,), jnp.float32),
            pltpu.SMEM((n + k + 8,), jnp.float32),
            pltpu.SMEM((n + k + 8,), jnp.float32),
            pltpu.SemaphoreType.DMA,
        ],
    )(thr, aw, gf)

    return jnp.reshape(out, (b, c, d, h, w))


# speedup vs baseline: 1.1095x; 1.0063x over previous
import functools

import jax
import jax.numpy as jnp
from jax.experimental import pallas as pl
from jax.experimental.pallas import tpu as pltpu

_SMOOTH = 1e-5
_INV_SQRT2 = 0.7071067811865476


def _stencil_kernel(thr_ref, aw_ref, x_ref, o_ref,
                    g_scr, tail_scr, s_ref, inter_ref, mask_ref, dinv_ref,
                    w_ref, copy_sem, *, n, c, k):
    u = pl.program_id(0)
    nu = n // k
    kc, h, w = x_ref.shape
    hb = 16 if h % 16 == 0 else h
    g = min(8, k)
    thr = thr_ref[0]
    aw = aw_ref[0]
    base = k * u


    window_copy = pltpu.make_async_copy(x_ref, g_scr.at[u % 2], copy_sem)

    @pl.when(u <= nu - 1)
    def _():
        window_copy.start()

    prev_ref = g_scr.at[(u + 1) % 2]

    def tile(ref, j, h0):
        return ref[j * c:(j + 1) * c, h0:h0 + hb, :]

    s_tot = [None] * k
    i_tot = [None] * k
    sub = 8 if (c * hb) % 8 == 0 else 1

    def psum(t3):
        return jnp.sum(jnp.reshape(t3, (-1, sub, w)), axis=0)

    cl = n + k

    def put_s(i, v):
        s_ref[jnp.clip(i, 0, cl)] = v

    def put_inter(i, v):
        inter_ref[jnp.clip(i, 0, cl)] = v

    def put_mask(i):
        ic = jnp.clip(i, 0, cl)
        dice = ((2.0 * inter_ref[ic] + _SMOOTH)
                / (s_ref[ic] + s_ref[jnp.clip(i + 1, 0, cl)] + _SMOOTH))
        mask_ref[ic] = jnp.where((dice > thr) & (dice < 1.0), 1.0, 0.0)

    def getm(i):
        ok = (i >= 0) & (i <= n - 2)
        return jnp.where(ok, mask_ref[jnp.clip(i, 0, n - 2)], 0.0)

    def put_dinv(m):
        deg = getm(m - 1) + getm(m)
        dinv_ref[jnp.clip(m, 0, cl)] = jnp.where(
            deg > 1.5, _INV_SQRT2, jnp.where(deg > 0.5, 1.0, 0.0))

    def put_w(i):
        ic = jnp.clip(i, 0, cl)
        w_ref[ic] = (aw * getm(i) * dinv_ref[ic]
                     * dinv_ref[jnp.clip(i + 1, 0, cl)])

    for g0 in range(0, k, g):
        s_acc = [jnp.zeros((sub, w), jnp.float32) for _ in range(g)]
        i_acc = [jnp.zeros((sub, w), jnp.float32) for _ in range(g)]
        for h0 in range(0, h, hb):
            if g0 == 0:
                pbl = (tile(prev_ref, k - 1, h0) > 0.0).astype(jnp.float32)
            else:
                pbl = (tile(x_ref, g0 - 1, h0) > 0.0).astype(jnp.float32)
            for t in range(g):
                v = tile(x_ref, g0 + t, h0)
                pb = (v > 0.0).astype(jnp.float32)
                s_acc[t] = s_acc[t] + psum(pb)
                i_acc[t] = i_acc[t] + psum(pb * pbl)
                pbl = pb
        for t in range(g):
            s_tot[g0 + t] = jnp.sum(s_acc[t])
            i_tot[g0 + t] = jnp.sum(i_acc[t])
            put_s(base + g0 + t, s_tot[g0 + t])
            put_inter(base + g0 + t - 1, i_tot[g0 + t])
        for off in range(g0 - 1, g0 + g - 1):
            put_mask(base + off)
        for off in range(g0 - 1, g0 + g - 1):
            put_dinv(base + off)
        for off in range(g0 - 2, g0 + g - 2):
            put_w(base + off)

    def getw(i):
        ok = (i >= 0) & (i <= n - 2)
        return jnp.where(ok, w_ref[jnp.clip(i, 0, n - 2)], 0.0)

    m0 = base - k
    wl = [getw(m0 + j - 1) for j in range(k)]
    wr = [getw(m0 + j) for j in range(k)]

    @pl.when(u >= 1)
    def _():
        for h0 in range(0, h, hb):
            r_prev = jnp.maximum(tail_scr[:, h0:h0 + hb, :], 0.0)
            v_cur = tile(prev_ref, 0, h0)
            r_cur = jnp.maximum(v_cur, 0.0)
            for j in range(k):
                if j <= k - 2:
                    v_next = tile(prev_ref, j + 1, h0)
                else:
                    v_next = tile(x_ref, 0, h0)
                r_next = jnp.maximum(v_next, 0.0)
                o_ref[j * c:(j + 1) * c, h0:h0 + hb, :] = v_cur + (
                    r_prev * wl[j] + r_next * wr[j])
                r_prev = r_cur
                v_cur = v_next
                r_cur = r_next

    @pl.when(u >= 1)
    def _():
        tail_scr[...] = prev_ref[(k - 1) * c:, :, :]

    @pl.when(u == 0)
    def _():
        tail_scr[...] = jnp.zeros_like(tail_scr)

    @pl.when(u <= nu - 1)
    def _():
        window_copy.wait()


def kernel(x, adaptive_weight, similarity_threshold):
    b, c, d, h, w = x.shape
    n = b * d
    k = 16
    while n % k != 0 or n // k < 2:
        k //= 2
    nu = n // k
    gf = jnp.reshape(x, (b * c * d, h, w))
    thr = jnp.reshape(similarity_threshold, (1,)).astype(jnp.float32)
    aw = jnp.reshape(adaptive_weight, (1,)).astype(jnp.float32)

    out = pl.pallas_call(
        functools.partial(_stencil_kernel, n=n, c=c, k=k),
        grid=(nu + 1,),
        in_specs=[
            pl.BlockSpec(memory_space=pltpu.SMEM),
            pl.BlockSpec(memory_space=pltpu.SMEM),
            pl.BlockSpec((k * c, h, w),
                         lambda u: (jnp.minimum(u, nu - 1), 0, 0)),
        ],
        out_specs=pl.BlockSpec((k * c, h, w),
                               lambda u: (jnp.clip(u - 1, 0, nu - 1), 0, 0)),
        out_shape=jax.ShapeDtypeStruct((b * c * d, h, w), jnp.float32),
        scratch_shapes=[
            pltpu.VMEM((2, k * c, h, w), jnp.float32),
            pltpu.VMEM((c, h, w), jnp.float32),
            pltpu.SMEM((n + k + 8,), jnp.float32),
            pltpu.SMEM((n + k + 8,), jnp.float32),
            pltpu.SMEM((n + k + 8,), jnp.float32),
            pltpu.SMEM((n + k + 8,), jnp.float32),
            pltpu.SMEM((n + k + 8,), jnp.float32),
            pltpu.SemaphoreType.DMA,
        ],
    )(thr, aw, gf)

    return jnp.reshape(out, (b, c, d, h, w))
